# baseline (speedup 1.0000x reference)
import functools

import jax
import jax.numpy as jnp
from jax import lax
from jax.experimental import pallas as pl
from jax.experimental.pallas import tpu as pltpu
from jax.experimental.pallas import tpu_sc as plsc

BATCH = 4096
SEQ = 200
DIM = 64

NC = 2
NS = 16
NW = NC * NS

ROWS = BATCH * SEQ
ROWS_PER_W = ROWS // NW
CHUNK = 128
CHUNKS_PER_W = ROWS_PER_W // CHUNK


def _body(seq_hbm, tok_hbm, pos_hbm, out_hbm, idx_v, buf_v, pos2_v, sem):
    wid = lax.axis_index("s") * NC + lax.axis_index("c")
    base = wid * ROWS_PER_W

    pltpu.sync_copy(pos_hbm, pos2_v.at[pl.ds(0, SEQ), :])
    pltpu.sync_copy(pos_hbm, pos2_v.at[pl.ds(SEQ, SEQ), :])

    def chunk_body(c, carry):
        row0 = base + c * CHUNK
        pltpu.sync_copy(seq_hbm.at[pl.ds(row0, CHUNK)], idx_v)
        pltpu.async_copy(tok_hbm.at[idx_v], buf_v, sem).wait()
        s0 = lax.rem(c * CHUNK, SEQ)

        def row_add(r, rcarry):
            for j in range(DIM // 16):
                sl = pl.ds(j * 16, 16)
                buf_v[r, sl] = buf_v[r, sl] + pos2_v[s0 + r, sl]
            return rcarry

        lax.fori_loop(0, CHUNK, row_add, 0, unroll=2)
        pltpu.sync_copy(buf_v, out_hbm.at[pl.ds(row0, CHUNK), :])
        return carry

    lax.fori_loop(0, CHUNKS_PER_W, chunk_body, 0)


def kernel(seq, token_table, pos_table):
    seq_flat = seq.reshape(ROWS)
    run = functools.partial(
        pl.kernel,
        out_type=jax.ShapeDtypeStruct((ROWS, DIM), jnp.float32),
        mesh=plsc.VectorSubcoreMesh(core_axis_name="c", subcore_axis_name="s"),
        scratch_types=[
            pltpu.VMEM((CHUNK,), jnp.int32),
            pltpu.VMEM((CHUNK, DIM), jnp.float32),
            pltpu.VMEM((2 * SEQ, DIM), jnp.float32),
            pltpu.SemaphoreType.DMA,
        ],
        compiler_params=pltpu.CompilerParams(use_tc_tiling_on_sc=False),
    )(_body)
    out = run(seq_flat, token_table, pos_table)
    return out.reshape(BATCH, SEQ, DIM)


# speedup vs baseline: 1.2318x; 1.2318x over previous
import functools

import jax
import jax.numpy as jnp
from jax import lax
from jax.experimental import pallas as pl
from jax.experimental.pallas import tpu as pltpu
from jax.experimental.pallas import tpu_sc as plsc

BATCH = 4096
SEQ = 200
DIM = 64

NC = 2
NS = 16
NW = NC * NS

ROWS = BATCH * SEQ
ROWS_PER_W = ROWS // NW
CHUNK = 128
CHUNKS_PER_W = ROWS_PER_W // CHUNK
NBANK = 3
GROUP = 2
NGROUPS = CHUNKS_PER_W // GROUP
NBUF = NBANK * GROUP


def _body(seq_hbm, tok_hbm, pos_hbm, out_hbm, idx_v, pos2_v, *rest):
    bufs = rest[:NBUF]
    sem_g = rest[NBUF:2 * NBUF]
    sem_w = rest[2 * NBUF:3 * NBUF]

    wid = lax.axis_index("s") * NC + lax.axis_index("c")
    base = wid * ROWS_PER_W

    pltpu.sync_copy(seq_hbm.at[pl.ds(base, ROWS_PER_W)], idx_v)
    pltpu.sync_copy(pos_hbm, pos2_v.at[pl.ds(0, SEQ), :])
    pltpu.sync_copy(pos_hbm, pos2_v.at[pl.ds(SEQ, SEQ), :])

    def fire_gather(b, c):
        pltpu.async_copy(tok_hbm.at[idx_v.at[pl.ds(c * CHUNK, CHUNK)]],
                         bufs[b], sem_g[b])

    def wait_gather(b):
        pltpu.make_async_copy(
            tok_hbm.at[idx_v.at[pl.ds(0, CHUNK)]], bufs[b], sem_g[b]).wait()

    def fire_write(b, c):
        pltpu.async_copy(bufs[b], out_hbm.at[pl.ds(base + c * CHUNK, CHUNK), :],
                         sem_w[b])

    def wait_write(b):
        pltpu.make_async_copy(
            bufs[b], out_hbm.at[pl.ds(base, CHUNK), :], sem_w[b]).wait()

    def add_pos(b, c):
        s0 = lax.rem(c * CHUNK, SEQ)
        buf = bufs[b]

        def row_add(r, rcarry):
            for j in range(DIM // 16):
                sl = pl.ds(j * 16, 16)
                buf[r, sl] = buf[r, sl] + pos2_v[s0 + r, sl]
            return rcarry

        lax.fori_loop(0, CHUNK, row_add, 0, unroll=4)

    for p in range(2):
        for j in range(GROUP):
            fire_gather(GROUP * p + j, GROUP * p + j)

    def outer(t, carry):
        for p in range(NBANK):
            g = NBANK * t + p
            for j in range(GROUP):
                b = GROUP * p + j
                c = GROUP * g + j
                wait_gather(b)
                add_pos(b, c)
                fire_write(b, c)
            q = (p + 2) % NBANK

            def fire_next():
                for j in range(GROUP):
                    b = GROUP * q + j
                    if p == 0:
                        @pl.when(t > 0)
                        def _():
                            wait_write(b)
                    else:
                        wait_write(b)
                    fire_gather(b, GROUP * (g + 2) + j)

            if p == NBANK - 1:
                @pl.when(t < (NGROUPS - 1) // NBANK - 1)
                def _():
                    fire_next()
            else:
                fire_next()
        return carry

    lax.fori_loop(0, (NGROUPS - 1) // NBANK, outer, 0)

    g = NGROUPS - 1
    for j in range(GROUP):
        b = j
        wait_gather(b)
        add_pos(b, GROUP * g + j)
        fire_write(b, GROUP * g + j)
    for b in range(NBUF):
        wait_write(b)


def kernel(seq, token_table, pos_table):
    seq_flat = seq.reshape(ROWS)
    run = functools.partial(
        pl.kernel,
        out_type=jax.ShapeDtypeStruct((ROWS, DIM), jnp.float32),
        mesh=plsc.VectorSubcoreMesh(core_axis_name="c", subcore_axis_name="s"),
        scratch_types=(
            [pltpu.VMEM((ROWS_PER_W,), jnp.int32),
             pltpu.VMEM((2 * SEQ, DIM), jnp.float32)]
            + [pltpu.VMEM((CHUNK, DIM), jnp.float32) for _ in range(NBUF)]
            + [pltpu.SemaphoreType.DMA for _ in range(2 * NBUF)]
        ),
        compiler_params=pltpu.CompilerParams(use_tc_tiling_on_sc=False),
    )(_body)
    out = run(seq_flat, token_table, pos_table)
    return out.reshape(BATCH, SEQ, DIM)


# speedup vs baseline: 1.5534x; 1.2611x over previous
import functools

import jax
import jax.numpy as jnp
from jax import lax
from jax.experimental import pallas as pl
from jax.experimental.pallas import tpu as pltpu
from jax.experimental.pallas import tpu_sc as plsc

BATCH = 4096
SEQ = 200
DIM = 64

NC = 2
NS = 16
NW = NC * NS

ROWS = BATCH * SEQ
ROWS_PER_W = ROWS // NW
CHUNK = 128
CHUNKS_PER_W = ROWS_PER_W // CHUNK
NBANK = 3
GROUP = 2
NGROUPS = CHUNKS_PER_W // GROUP
NBUF = NBANK * GROUP


def _body(seq_hbm, tok_hbm, pos_hbm, out_hbm, idx_v, pos2_v, *rest):
    bufs = rest[:NBUF]
    sem_g = rest[NBUF:2 * NBUF]
    sem_w = rest[2 * NBUF:3 * NBUF]

    wid = lax.axis_index("s") * NC + lax.axis_index("c")
    base = wid * ROWS_PER_W

    pltpu.sync_copy(seq_hbm.at[pl.ds(base, ROWS_PER_W)], idx_v)
    pltpu.sync_copy(pos_hbm, pos2_v.at[pl.ds(0, SEQ), :])
    pltpu.sync_copy(pos_hbm, pos2_v.at[pl.ds(SEQ, SEQ), :])

    def fire_gather(b, c):
        pltpu.async_copy(tok_hbm.at[idx_v.at[pl.ds(c * CHUNK, CHUNK)]],
                         bufs[b], sem_g[b])

    def wait_gather(b):
        pltpu.make_async_copy(
            tok_hbm.at[idx_v.at[pl.ds(0, CHUNK)]], bufs[b], sem_g[b]).wait()

    def fire_write(b, c):
        pltpu.async_copy(bufs[b], out_hbm.at[pl.ds(base + c * CHUNK, CHUNK), :],
                         sem_w[b])

    def wait_write(b):
        pltpu.make_async_copy(
            bufs[b], out_hbm.at[pl.ds(base, CHUNK), :], sem_w[b]).wait()

    def add_pos(b, c):
        s0 = lax.rem(c * CHUNK, SEQ)
        buf = bufs[b]

        @plsc.parallel_loop(0, CHUNK, unroll=4)
        def _(r):
            for j in range(DIM // 16):
                sl = pl.ds(j * 16, 16)
                buf[r, sl] = buf[r, sl] + pos2_v[s0 + r, sl]

    for p in range(2):
        for j in range(GROUP):
            fire_gather(GROUP * p + j, GROUP * p + j)

    def outer(t, carry):
        for p in range(NBANK):
            g = NBANK * t + p
            for j in range(GROUP):
                b = GROUP * p + j
                c = GROUP * g + j
                wait_gather(b)
                add_pos(b, c)
                fire_write(b, c)
            q = (p + 2) % NBANK

            def fire_next():
                for j in range(GROUP):
                    b = GROUP * q + j
                    if p == 0:
                        @pl.when(t > 0)
                        def _():
                            wait_write(b)
                    else:
                        wait_write(b)
                    fire_gather(b, GROUP * (g + 2) + j)

            if p == NBANK - 1:
                @pl.when(t < (NGROUPS - 1) // NBANK - 1)
                def _():
                    fire_next()
            else:
                fire_next()
        return carry

    lax.fori_loop(0, (NGROUPS - 1) // NBANK, outer, 0)

    g = NGROUPS - 1
    for j in range(GROUP):
        b = j
        wait_gather(b)
        add_pos(b, GROUP * g + j)
        fire_write(b, GROUP * g + j)
    for b in range(NBUF):
        wait_write(b)


def kernel(seq, token_table, pos_table):
    seq_flat = seq.reshape(ROWS)
    run = functools.partial(
        pl.kernel,
        out_type=jax.ShapeDtypeStruct((ROWS, DIM), jnp.float32),
        mesh=plsc.VectorSubcoreMesh(core_axis_name="c", subcore_axis_name="s"),
        scratch_types=(
            [pltpu.VMEM((ROWS_PER_W,), jnp.int32),
             pltpu.VMEM((2 * SEQ, DIM), jnp.float32)]
            + [pltpu.VMEM((CHUNK, DIM), jnp.float32) for _ in range(NBUF)]
            + [pltpu.SemaphoreType.DMA for _ in range(2 * NBUF)]
        ),
        compiler_params=pltpu.CompilerParams(use_tc_tiling_on_sc=False),
    )(_body)
    out = run(seq_flat, token_table, pos_table)
    return out.reshape(BATCH, SEQ, DIM)
